# baseline (speedup 1.0000x reference)
import functools

import numpy as np
import jax
import jax.numpy as jnp
from jax import lax
from jax.experimental import pallas as pl
from jax.experimental.pallas import tpu as pltpu
from jax.experimental.pallas import tpu_sc as plsc

_B, _S, _D = 4096, 200, 64
_N = _B * _S
_NC, _NS, _L = 2, 16, 16
_NW = _NC * _NS
_PER_W = _N // _NW
_CHUNK = 128
_NCHUNK = _PER_W // _CHUNK
_NBUF = 2


def _pos_enc8() -> np.ndarray:
    d = np.arange(_D)[np.newaxis, :]
    d = 1.0 / np.power(10000, 2 * (d // 2) / np.float32(_D))
    t = np.arange(_S)[:, np.newaxis] * d
    t = np.concatenate([np.sin(t[:, 0::2]), np.cos(t[:, 1::2])], axis=-1)
    return (t * 8.0).astype(np.float32).reshape(-1)


def _make_kernel():
    mesh = plsc.VectorSubcoreMesh(
        core_axis_name="c", subcore_axis_name="s",
        num_cores=_NC, num_subcores=_NS,
    )

    @functools.partial(
        pl.kernel,
        out_type=jax.ShapeDtypeStruct((_N, _D), jnp.float32),
        mesh=mesh,
        scratch_types=[
            pltpu.VMEM((_PER_W,), jnp.int32),
            pltpu.VMEM((_S * _D,), jnp.float32),
            pltpu.VMEM((_NBUF, _CHUNK, 2 * _D), jnp.float32),
            pltpu.VMEM((_NBUF, _CHUNK, _D), jnp.float32),
            pltpu.SemaphoreType.DMA,
            pltpu.SemaphoreType.DMA,
        ],
        compiler_params=pltpu.CompilerParams(use_tc_tiling_on_sc=True),
    )
    def body(y_hbm, pos_hbm, emb_hbm, out_hbm, idx_v, pos_v, buf_v, o_v,
             sem0, sem1):
        sems = (sem0, sem1)
        wid = lax.axis_index("s") * _NC + lax.axis_index("c")
        row0 = wid * _PER_W
        pltpu.sync_copy(y_hbm.at[pl.ds(row0, _PER_W)], idx_v)
        pltpu.sync_copy(pos_hbm, pos_v)

        def start(cc, b):
            pltpu.async_copy(
                emb_hbm.at[idx_v.at[pl.ds(cc * _CHUNK, _CHUNK)]],
                buf_v.at[b], sems[b])

        def wait(cc, b):
            pltpu.make_async_copy(
                emb_hbm.at[idx_v.at[pl.ds(cc * _CHUNK, _CHUNK)]],
                buf_v.at[b], sems[b]).wait()

        for b in range(_NBUF):
            start(b, b)

        @pl.loop(0, _NCHUNK, step=_NBUF)
        def _chunks(c):
            for b in range(_NBUF):
                cc = c + b
                wait(cc, b)

                @pl.loop(0, _CHUNK)
                def _rows(r):
                    s = lax.rem(cc * _CHUNK + r, _S)
                    for k in range(_D // _L):
                        sl = pl.ds(k * _L, _L)
                        o_v[b, r, sl] = (buf_v[b, r, sl] * 8.0
                                         + pos_v[pl.ds(s * _D + k * _L, _L)])

                pltpu.sync_copy(
                    o_v.at[b], out_hbm.at[pl.ds(row0 + cc * _CHUNK, _CHUNK)])

                nxt = cc + _NBUF

                @pl.when(nxt < _NCHUNK)
                def _():
                    start(nxt, b)

    return body


_EMBED_KERNEL = _make_kernel()
_POS8 = _pos_enc8()


def kernel(y, lens, emb):
    yflat = y.reshape(_N)
    embp = jnp.pad(emb, ((0, 0), (0, _D)))
    out = _EMBED_KERNEL(yflat, jnp.asarray(_POS8), embp)
    return out.reshape(_B, _S, _D), lens
